# Initial kernel scaffold; baseline (speedup 1.0000x reference)
import jax
import jax.numpy as jnp
from jax.experimental import pallas as pl


def kernel(u, labels, w):
    raise NotImplementedError("write your pallas kernel here")


# speedup vs baseline: 1.2772x; 1.2772x over previous
import functools

import jax
import jax.numpy as jnp
from jax import lax
from jax.experimental import pallas as pl
from jax.experimental.pallas import tpu as pltpu

_LMBD0 = 0.001
_LMBD1 = 1.0
_LMBD2 = 0.001
_ALPHA = 1.0


def _cdiv(a, b):
    return -(-a // b)


def _agg_kernel(lbl_ref, u_ref, out_ref, acc_ref, *, num_classes):
    r = pl.program_id(1)

    @pl.when(r == 0)
    def _():
        acc_ref[...] = jnp.zeros_like(acc_ref)

    t = jnp.tanh(u_ref[...].astype(jnp.float32))
    tn, nbit = t.shape
    rowsq = jnp.sum(t * t, axis=1, keepdims=True)
    rowabs = jnp.sum(jnp.abs(t), axis=1, keepdims=True)

    lane = lax.broadcasted_iota(jnp.int32, (tn, nbit), 1)
    aux = jnp.where(lane == 0, rowsq,
                    jnp.where(lane == 1, rowabs,
                              jnp.where(lane == 2, 1.0, 0.0)))
    rhs = jnp.concatenate([t, aux], axis=1).astype(jnp.bfloat16)

    lbl = lbl_ref[...]
    cio = lax.broadcasted_iota(jnp.int32, (tn, num_classes), 1)
    lhs = (cio == lbl).astype(jnp.bfloat16)

    acc_ref[...] += lax.dot_general(
        lhs, rhs, (((0,), (0,)), ((), ())),
        preferred_element_type=jnp.float32)

    @pl.when(r == pl.num_programs(1) - 1)
    def _():
        out_ref[0] = acc_ref[...]


def kernel(u, labels, w):
    n, nbit = u.shape
    c = w.shape[1]
    num_splits = 2
    tn = 1024
    rows_per_split = _cdiv(n, num_splits)
    r_blocks = _cdiv(rows_per_split, tn)
    padded = num_splits * r_blocks * tn

    lbl = labels.reshape(n, 1).astype(jnp.int32)
    u_in = u
    if padded != n:
        u_in = jnp.zeros((padded, nbit), u.dtype).at[:n].set(u)
        lbl = jnp.full((padded, 1), c, jnp.int32).at[:n].set(lbl)

    width = 2 * nbit
    parts = pl.pallas_call(
        functools.partial(_agg_kernel, num_classes=c),
        out_shape=jax.ShapeDtypeStruct((num_splits, c, width), jnp.float32),
        grid_spec=pltpu.PrefetchScalarGridSpec(
            num_scalar_prefetch=0,
            grid=(num_splits, r_blocks),
            in_specs=[
                pl.BlockSpec((tn, 1), lambda s, r: (s * r_blocks + r, 0)),
                pl.BlockSpec((tn, nbit), lambda s, r: (s * r_blocks + r, 0)),
            ],
            out_specs=pl.BlockSpec((1, c, width), lambda s, r: (s, 0, 0)),
            scratch_shapes=[pltpu.VMEM((c, width), jnp.float32)],
        ),
        compiler_params=pltpu.CompilerParams(
            dimension_semantics=("parallel", "arbitrary"),
            vmem_limit_bytes=64 * 1024 * 1024,
        ),
    )(lbl, u_in)

    agg = parts[0] + parts[1]
    m = agg[:, :nbit]
    t_c = agg[:, nbit]
    sum_abs = jnp.sum(agg[:, nbit + 1])
    counts = agg[:, nbit + 2]
    ucol = jnp.sum(m, axis=0)
    sum_uu = jnp.sum(t_c)

    n_f = float(n)
    j1 = (sum_uu - 2.0 * sum_abs + n_f * float(nbit)) / n_f
    j2_1 = sum_uu
    posn = jnp.sum(counts * counts)
    negn = n_f * n_f - posn
    s_pos = 2.0 * jnp.sum(counts * t_c) - 2.0 * jnp.sum(m * m)
    s_all = 2.0 * n_f * sum_uu - 2.0 * jnp.sum(ucol * ucol)
    s_neg = s_all - s_pos
    j2_2 = s_neg / (negn + 1e-7) - s_pos / (posn + 1e-7)
    j2 = (j2_1 + _ALPHA * j2_2) / 2.0

    wf = w.astype(jnp.float32)
    ortho = wf @ wf.T
    j3 = jnp.sum((ortho - 1.0) ** 2) / 2.0

    return _LMBD0 * j1 - _LMBD1 * j2 + _LMBD2 * j3


# speedup vs baseline: 1.8161x; 1.4219x over previous
import functools

import jax
import jax.numpy as jnp
from jax import lax
from jax.experimental import pallas as pl
from jax.experimental.pallas import tpu as pltpu

_LMBD0 = 0.001
_LMBD1 = 1.0
_LMBD2 = 0.001
_ALPHA = 1.0


def _cdiv(a, b):
    return -(-a // b)


def _agg_kernel(lbl_ref, u_ref, out_ref, acc_ref, *, num_classes):
    r = pl.program_id(1)

    @pl.when(r == 0)
    def _():
        acc_ref[...] = jnp.zeros_like(acc_ref)

    t = jnp.tanh(u_ref[...].astype(jnp.float32))
    tn, nbit = t.shape
    rowsq = jnp.sum(t * t, axis=1, keepdims=True)
    rowabs = jnp.sum(jnp.abs(t), axis=1, keepdims=True)

    lane = lax.broadcasted_iota(jnp.int32, (tn, nbit), 1)
    aux = jnp.where(lane == 0, rowsq,
                    jnp.where(lane == 1, rowabs,
                              jnp.where(lane == 2, 1.0, 0.0)))
    rhs = jnp.concatenate([t, aux], axis=1).astype(jnp.bfloat16)

    lbl = lbl_ref[...]
    cio = lax.broadcasted_iota(jnp.int32, (tn, num_classes), 1)
    lhs = (cio == lbl).astype(jnp.bfloat16)

    acc_ref[...] += lax.dot_general(
        lhs, rhs, (((0,), (0,)), ((), ())),
        preferred_element_type=jnp.float32)

    @pl.when(r == pl.num_programs(1) - 1)
    def _():
        out_ref[0] = acc_ref[...]


def _finalize_kernel(parts_ref, w_ref, out_ref, *, n, nbit):
    agg = parts_ref[0] + parts_ref[1]
    m = agg[:, :nbit]
    t_c = agg[:, nbit:nbit + 1]
    sum_abs = jnp.sum(agg[:, nbit + 1:nbit + 2])
    counts = agg[:, nbit + 2:nbit + 3]
    ucol = jnp.sum(m, axis=0, keepdims=True)
    sum_uu = jnp.sum(t_c)

    n_f = float(n)
    j1 = (sum_uu - 2.0 * sum_abs + n_f * float(nbit)) / n_f
    posn = jnp.sum(counts * counts)
    negn = n_f * n_f - posn
    s_pos = 2.0 * jnp.sum(counts * t_c) - 2.0 * jnp.sum(m * m)
    s_all = 2.0 * n_f * sum_uu - 2.0 * jnp.sum(ucol * ucol)
    s_neg = s_all - s_pos
    j2_2 = s_neg / (negn + 1e-7) - s_pos / (posn + 1e-7)
    j2 = (sum_uu + _ALPHA * j2_2) / 2.0

    wf = w_ref[...]
    ortho = lax.dot_general(wf, wf, (((1,), (1,)), ((), ())),
                            preferred_element_type=jnp.float32)
    j3 = jnp.sum((ortho - 1.0) ** 2) / 2.0

    loss = _LMBD0 * j1 - _LMBD1 * j2 + _LMBD2 * j3
    out_ref[...] = jnp.reshape(loss, (1, 1))


def kernel(u, labels, w):
    n, nbit = u.shape
    c = w.shape[1]
    num_splits = 2
    tn = 2048
    rows_per_split = _cdiv(n, num_splits)
    r_blocks = _cdiv(rows_per_split, tn)
    padded = num_splits * r_blocks * tn

    lbl = labels.reshape(n, 1).astype(jnp.int32)
    u_in = u
    if padded != n:
        u_in = jnp.zeros((padded, nbit), u.dtype).at[:n].set(u)
        lbl = jnp.full((padded, 1), c, jnp.int32).at[:n].set(lbl)

    width = 2 * nbit
    parts = pl.pallas_call(
        functools.partial(_agg_kernel, num_classes=c),
        out_shape=jax.ShapeDtypeStruct((num_splits, c, width), jnp.float32),
        grid_spec=pltpu.PrefetchScalarGridSpec(
            num_scalar_prefetch=0,
            grid=(num_splits, r_blocks),
            in_specs=[
                pl.BlockSpec((tn, 1), lambda s, r: (s * r_blocks + r, 0)),
                pl.BlockSpec((tn, nbit), lambda s, r: (s * r_blocks + r, 0)),
            ],
            out_specs=pl.BlockSpec((1, c, width), lambda s, r: (s, 0, 0)),
            scratch_shapes=[pltpu.VMEM((c, width), jnp.float32)],
        ),
        compiler_params=pltpu.CompilerParams(
            dimension_semantics=("parallel", "arbitrary"),
            vmem_limit_bytes=64 * 1024 * 1024,
        ),
    )(lbl, u_in)

    out = pl.pallas_call(
        functools.partial(_finalize_kernel, n=n, nbit=nbit),
        out_shape=jax.ShapeDtypeStruct((1, 1), jnp.float32),
    )(parts, w.astype(jnp.float32))
    return out[0, 0]


# speedup vs baseline: 2.0886x; 1.1500x over previous
import functools

import jax
import jax.numpy as jnp
from jax import lax
from jax.experimental import pallas as pl
from jax.experimental.pallas import tpu as pltpu

_LMBD0 = 0.001
_LMBD1 = 1.0
_LMBD2 = 0.001
_ALPHA = 1.0


def _cdiv(a, b):
    return -(-a // b)


def _contrib(lbl, u, num_classes):
    t = jnp.tanh(u)
    tn, nbit = t.shape
    rowsq = jnp.sum(t * t, axis=1, keepdims=True)
    rowabs = jnp.sum(jnp.abs(t), axis=1, keepdims=True)

    lane = lax.broadcasted_iota(jnp.int32, (tn, nbit), 1)
    aux = jnp.where(lane == 0, rowsq,
                    jnp.where(lane == 1, rowabs,
                              jnp.where(lane == 2, 1.0, 0.0)))
    rhs = jnp.concatenate([t, aux], axis=1).astype(jnp.bfloat16)

    cio = lax.broadcasted_iota(jnp.int32, (tn, num_classes), 1)
    lhs = (cio == lbl).astype(jnp.bfloat16)

    return lax.dot_general(
        lhs, rhs, (((0,), (0,)), ((), ())),
        preferred_element_type=jnp.float32)


def _agg_kernel(*refs, num_classes, nstream):
    lbl_refs = refs[:nstream]
    u_refs = refs[nstream:2 * nstream]
    out_ref = refs[2 * nstream]
    acc_ref = refs[2 * nstream + 1]
    r = pl.program_id(1)

    @pl.when(r == 0)
    def _():
        acc_ref[...] = jnp.zeros_like(acc_ref)

    total = _contrib(lbl_refs[0][...], u_refs[0][...], num_classes)
    for k in range(1, nstream):
        total += _contrib(lbl_refs[k][...], u_refs[k][...], num_classes)
    acc_ref[...] += total

    @pl.when(r == pl.num_programs(1) - 1)
    def _():
        out_ref[0] = acc_ref[...]


def _finalize_kernel(parts_ref, w_ref, out_ref, *, n, nbit):
    agg = parts_ref[0] + parts_ref[1]
    m = agg[:, :nbit]
    t_c = agg[:, nbit:nbit + 1]
    sum_abs = jnp.sum(agg[:, nbit + 1:nbit + 2])
    counts = agg[:, nbit + 2:nbit + 3]
    ucol = jnp.sum(m, axis=0, keepdims=True)
    sum_uu = jnp.sum(t_c)

    n_f = float(n)
    j1 = (sum_uu - 2.0 * sum_abs + n_f * float(nbit)) / n_f
    posn = jnp.sum(counts * counts)
    negn = n_f * n_f - posn
    s_pos = 2.0 * jnp.sum(counts * t_c) - 2.0 * jnp.sum(m * m)
    s_all = 2.0 * n_f * sum_uu - 2.0 * jnp.sum(ucol * ucol)
    s_neg = s_all - s_pos
    j2_2 = s_neg / (negn + 1e-7) - s_pos / (posn + 1e-7)
    j2 = (sum_uu + _ALPHA * j2_2) / 2.0

    wf = w_ref[...]
    ortho = lax.dot_general(wf, wf, (((1,), (1,)), ((), ())),
                            preferred_element_type=jnp.float32)
    j3 = jnp.sum((ortho - 1.0) ** 2) / 2.0

    loss = _LMBD0 * j1 - _LMBD1 * j2 + _LMBD2 * j3
    out_ref[...] = jnp.reshape(loss, (1, 1))


def kernel(u, labels, w):
    n, nbit = u.shape
    c = w.shape[1]
    num_splits = 2
    tn = 2048
    nstream = 2
    rows_per_step = tn * nstream
    rows_per_split = _cdiv(n, num_splits)
    r_blocks = _cdiv(rows_per_split, rows_per_step)
    padded = num_splits * r_blocks * rows_per_step

    lbl = labels.reshape(n, 1).astype(jnp.int32)
    u_in = u
    if padded != n:
        u_in = jnp.zeros((padded, nbit), u.dtype).at[:n].set(u)
        lbl = jnp.full((padded, 1), c, jnp.int32).at[:n].set(lbl)

    width = 2 * nbit
    lbl_specs = [
        pl.BlockSpec((tn, 1),
                     lambda s, r, kk=k: ((s * r_blocks + r) * nstream + kk, 0))
        for k in range(nstream)
    ]
    u_specs = [
        pl.BlockSpec((tn, nbit),
                     lambda s, r, kk=k: ((s * r_blocks + r) * nstream + kk, 0))
        for k in range(nstream)
    ]
    parts = pl.pallas_call(
        functools.partial(_agg_kernel, num_classes=c, nstream=nstream),
        out_shape=jax.ShapeDtypeStruct((num_splits, c, width), jnp.float32),
        grid_spec=pltpu.PrefetchScalarGridSpec(
            num_scalar_prefetch=0,
            grid=(num_splits, r_blocks),
            in_specs=lbl_specs + u_specs,
            out_specs=pl.BlockSpec((1, c, width), lambda s, r: (s, 0, 0)),
            scratch_shapes=[pltpu.VMEM((c, width), jnp.float32)],
        ),
        compiler_params=pltpu.CompilerParams(
            dimension_semantics=("parallel", "arbitrary"),
            vmem_limit_bytes=64 * 1024 * 1024,
        ),
    )(*([lbl] * nstream + [u_in] * nstream))

    out = pl.pallas_call(
        functools.partial(_finalize_kernel, n=n, nbit=nbit),
        out_shape=jax.ShapeDtypeStruct((1, 1), jnp.float32),
    )(parts, w.astype(jnp.float32))
    return out[0, 0]
